# baseline (speedup 1.0000x reference)
import functools

import jax
import jax.numpy as jnp
from jax import lax
from jax.experimental import pallas as pl
from jax.experimental.pallas import tpu as pltpu
from jax.experimental.pallas import tpu_sc as plsc

N = 10000
E = 320000
VOCAB = 100000
NW = 128
NH = 512
NT = 128
B = 512

NWORK = 32
NPAD = 10240
RPW = NPAD // NWORK
FPW = NW // NWORK
CHUNK = 3200
NCH = E // CHUNK

def _wid():
    return lax.axis_index("s") * 2 + lax.axis_index("c")


@functools.cache
def _vocab_gather_kernel():
    mesh = plsc.VectorSubcoreMesh(core_axis_name="c", subcore_axis_name="s")

    @functools.partial(
        pl.kernel,
        out_type=jax.ShapeDtypeStruct((NPAD, NW), jnp.float32),
        mesh=mesh,
        scratch_types=[
            pltpu.VMEM((RPW,), jnp.int32),
            pltpu.VMEM((RPW, NW), jnp.float32),
            pltpu.SemaphoreType.DMA,
        ],
    )
    def _vocab_gather(idx_hbm, table_hbm, out_hbm, idx_v, rows_v, sem):
        base = _wid() * RPW
        pltpu.sync_copy(idx_hbm.at[pl.ds(base, RPW)], idx_v)
        pltpu.async_copy(table_hbm.at[idx_v], rows_v, sem).wait()
        pltpu.sync_copy(rows_v, out_hbm.at[pl.ds(base, RPW)])

    return _vocab_gather


@functools.cache
def _edge_aggr_kernel():
    mesh = plsc.VectorSubcoreMesh(core_axis_name="c", subcore_axis_name="s")

    @functools.partial(
        pl.kernel,
        out_type=jax.ShapeDtypeStruct((NW, NPAD), jnp.float32),
        mesh=mesh,
        scratch_types=[
            pltpu.VMEM((FPW * NPAD,), jnp.float32),
            pltpu.VMEM((FPW * NPAD,), jnp.float32),
            pltpu.VMEM((2, CHUNK), jnp.int32),
            pltpu.VMEM((2, CHUNK), jnp.int32),
            pltpu.SemaphoreType.DMA,
            pltpu.SemaphoreType.DMA,
        ],
        compiler_params=pltpu.CompilerParams(
            needs_layout_passes=False, use_tc_tiling_on_sc=False),
    )
    def _edge_aggr(ed_hbm, xT_hbm, out_hbm, xv, av, eb0, eb1, sem0, sem1):
        wid = _wid()
        for f in range(FPW):
            pltpu.sync_copy(xT_hbm.at[wid * FPW + f],
                            xv.at[pl.ds(f * NPAD, NPAD)])

        @plsc.parallel_loop(0, FPW * NPAD // 16, 1, unroll=8)
        def _zero(i):
            av[pl.ds(i * 16, 16)] = jnp.zeros((16,), jnp.float32)

        def process(eb):
            @plsc.parallel_loop(0, CHUNK // 16, 2, unroll=8)
            def _grp(g):
                for u in range(2):
                    base = (g + u) * 16
                    sd = eb[0, pl.ds(base, 16)]
                    sg = lax.shift_right_logical(sd, 14)
                    dg = jnp.bitwise_and(sd, 16383)
                    wg = plsc.bitcast(eb[1, pl.ds(base, 16)], jnp.float32)
                    for f in range(FPW):
                        v = plsc.load_gather(xv, [sg + f * NPAD])
                        plsc.addupdate_scatter(av, [dg + f * NPAD], v * wg)

        pltpu.make_async_copy(ed_hbm.at[0], eb0, sem0).start()

        def pair(p, carry):
            ci = p * 2
            pltpu.make_async_copy(ed_hbm.at[ci], eb0, sem0).wait()
            pltpu.make_async_copy(ed_hbm.at[ci + 1], eb1, sem1).start()
            process(eb0)
            pltpu.make_async_copy(ed_hbm.at[ci + 1], eb1, sem1).wait()

            @pl.when(ci + 2 < NCH)
            def _():
                pltpu.make_async_copy(ed_hbm.at[ci + 2], eb0, sem0).start()

            process(eb1)
            return carry

        lax.fori_loop(0, NCH // 2, pair, 0)
        for f in range(FPW):
            pltpu.sync_copy(av.at[pl.ds(f * NPAD, NPAD)],
                            out_hbm.at[wid * FPW + f])

    return _edge_aggr


def _mask_vec():
    return (lax.broadcasted_iota(jnp.int32, (NPAD, 1), 0)
            < N).astype(jnp.float32)


def _d1_body(xT_ref, aggrT_ref, idxw_ref, Wrel_ref, brel_ref, Wroot_ref,
             g_ref, b_ref, out_ref):
    xT = xT_ref[...]
    aggT = aggrT_ref[...] + xT * idxw_ref[...]
    hT = (jnp.dot(Wrel_ref[...], aggT, preferred_element_type=jnp.float32)
          + brel_ref[...]
          + jnp.dot(Wroot_ref[...], xT, preferred_element_type=jnp.float32))
    mvec = _mask_vec()
    mu = jnp.dot(hT, mvec, preferred_element_type=jnp.float32) / N
    dev = hT - mu
    var = jnp.dot(dev * dev, mvec, preferred_element_type=jnp.float32) / N
    hn = dev * lax.rsqrt(var + 1e-5) * g_ref[...] + b_ref[...]
    out_ref[0:NW, :] = jnp.tanh(hn)
    out_ref[NW:2 * NW, :] = xT


_CB = 2048


def _d23_body(e1_ref, W1_ref, b1_ref, W2_ref, b2_ref, xb_ref, out_ref):
    k = pl.program_id(0)
    e1 = e1_ref[...]
    a = jnp.dot(W1_ref[...], e1, preferred_element_type=jnp.float32) + b1_ref[...]
    c = jnp.dot(W2_ref[...], e1, preferred_element_type=jnp.float32) + b2_ref[...]
    e2 = jax.nn.sigmoid(a) * jnp.tanh(c)
    sel = (xb_ref[...].reshape(_CB, 1)
           == lax.broadcasted_iota(jnp.int32, (_CB, B), 1)).astype(jnp.float32)
    p = jnp.dot(e2, sel, preferred_element_type=jnp.float32)

    @pl.when(k == 0)
    def _():
        out_ref[...] = p

    @pl.when(k > 0)
    def _():
        out_ref[...] += p


def _d4_body(e1_ref, xb_ref, e2s_ref, Wm_ref, bm_ref, Wl_ref, bl_ref,
             Wpa_ref, Wpb_ref, bp_ref, bg_ref, bb_ref,
             meanT_ref, logvarT_ref, phiT_ref):
    e2s = e2s_ref[...]
    meanT_ref[...] = (jnp.dot(Wm_ref[...], e2s, preferred_element_type=jnp.float32)
                      + bm_ref[...])
    logvarT_ref[...] = (jnp.dot(Wl_ref[...], e2s, preferred_element_type=jnp.float32)
                        + bl_ref[...])
    q = jnp.dot(Wpb_ref[...], e2s, preferred_element_type=jnp.float32)
    sel = (xb_ref[...].reshape(NPAD, 1)
           == lax.broadcasted_iota(jnp.int32, (NPAD, B), 1)).astype(jnp.float32)
    plT = (jnp.dot(Wpa_ref[...], e1_ref[...], preferred_element_type=jnp.float32)
           + lax.dot_general(q, sel, (((1,), (1,)), ((), ())),
                             preferred_element_type=jnp.float32)
           + bp_ref[...])
    mvec = _mask_vec()
    mup = jnp.dot(plT, mvec, preferred_element_type=jnp.float32) / N
    dev = plT - mup
    varp = jnp.dot(dev * dev, mvec, preferred_element_type=jnp.float32) / N
    pln = dev * lax.rsqrt(varp + 1e-5) * bg_ref[...] + bb_ref[...]
    m = jnp.max(pln, axis=0, keepdims=True)
    ex = jnp.exp(pln - m)
    phiT_ref[...] = ex / jnp.sum(ex, axis=0, keepdims=True)


def kernel(idx_x, idx_w, x_batch, edge_index, edge_weight, word_vec, W_rel,
           b_rel, W_root, bn1_g, bn1_b, W_fc1, b_fc1, W_fc2, b_fc2, W_mean,
           b_mean, W_logvar, b_logvar, W_phi, b_phi, bnp_g, bnp_b):
    f32 = jnp.float32
    idx_pad = jnp.concatenate(
        [idx_x.astype(jnp.int32), jnp.zeros((NPAD - N,), jnp.int32)])
    x_pad = _vocab_gather_kernel()(idx_pad, word_vec)
    xT_pad = x_pad.T

    src = edge_index[0].astype(jnp.int32)
    dst = edge_index[1].astype(jnp.int32)
    ew_bits = lax.bitcast_convert_type(edge_weight, jnp.int32)
    ed = (jnp.stack([src * 16384 + dst, ew_bits], axis=0)
          .reshape(2, NCH, CHUNK).transpose(1, 0, 2))
    aggrT_pad = _edge_aggr_kernel()(ed, xT_pad)

    idxw_pad = jnp.pad(idx_w, (0, NPAD - N)).reshape(1, NPAD)
    xb_pad = jnp.pad(x_batch.astype(jnp.int32), (0, NPAD - N),
                     constant_values=B).reshape(1, NPAD)

    enc1T = pl.pallas_call(
        _d1_body,
        out_shape=jax.ShapeDtypeStruct((2 * NW, NPAD), f32),
    )(xT_pad, aggrT_pad, idxw_pad, W_rel, b_rel.reshape(NW, 1), W_root,
      bn1_g.reshape(NW, 1), bn1_b.reshape(NW, 1))

    ncb = NPAD // _CB
    enc2sT = pl.pallas_call(
        _d23_body,
        grid=(ncb,),
        in_specs=[
            pl.BlockSpec((2 * NW, _CB), lambda k: (0, k)),
            pl.BlockSpec((NH, 2 * NW), lambda k: (0, 0)),
            pl.BlockSpec((NH, 1), lambda k: (0, 0)),
            pl.BlockSpec((NH, 2 * NW), lambda k: (0, 0)),
            pl.BlockSpec((NH, 1), lambda k: (0, 0)),
            pl.BlockSpec((1, _CB), lambda k: (0, k)),
        ],
        out_specs=pl.BlockSpec((NH, B), lambda k: (0, 0)),
        out_shape=jax.ShapeDtypeStruct((NH, B), f32),
    )(enc1T, W_fc1, b_fc1.reshape(NH, 1), W_fc2, b_fc2.reshape(NH, 1),
      xb_pad)

    meanT, logvarT, phiT = pl.pallas_call(
        _d4_body,
        out_shape=(
            jax.ShapeDtypeStruct((NT, B), f32),
            jax.ShapeDtypeStruct((NT, B), f32),
            jax.ShapeDtypeStruct((NT, NPAD), f32),
        ),
    )(enc1T, xb_pad, enc2sT, W_mean, b_mean.reshape(NT, 1), W_logvar,
      b_logvar.reshape(NT, 1), W_phi[:, :2 * NW], W_phi[:, 2 * NW:],
      b_phi.reshape(NT, 1), bnp_g.reshape(NT, 1), bnp_b.reshape(NT, 1))

    return (meanT.T, logvarT.T, phiT[:, :N].T)


# speedup vs baseline: 1.0812x; 1.0812x over previous
import functools

import jax
import jax.numpy as jnp
from jax import lax
from jax.experimental import pallas as pl
from jax.experimental.pallas import tpu as pltpu
from jax.experimental.pallas import tpu_sc as plsc

N = 10000
E = 320000
VOCAB = 100000
NW = 128
NH = 512
NT = 128
B = 512

NWORK = 32
NPAD = 10240
RPW = NPAD // NWORK
FPW = NW // NWORK
CHUNK = 3200
NCH = E // CHUNK

def _wid():
    return lax.axis_index("s") * 2 + lax.axis_index("c")


@functools.cache
def _vocab_gather_kernel():
    mesh = plsc.VectorSubcoreMesh(core_axis_name="c", subcore_axis_name="s")

    @functools.partial(
        pl.kernel,
        out_type=jax.ShapeDtypeStruct((NPAD, NW), jnp.float32),
        mesh=mesh,
        scratch_types=[
            pltpu.VMEM((RPW,), jnp.int32),
            pltpu.VMEM((RPW, NW), jnp.float32),
            pltpu.SemaphoreType.DMA,
        ],
    )
    def _vocab_gather(idx_hbm, table_hbm, out_hbm, idx_v, rows_v, sem):
        base = _wid() * RPW
        pltpu.sync_copy(idx_hbm.at[pl.ds(base, RPW)], idx_v)
        pltpu.async_copy(table_hbm.at[idx_v], rows_v, sem).wait()
        pltpu.sync_copy(rows_v, out_hbm.at[pl.ds(base, RPW)])

    return _vocab_gather


@functools.cache
def _edge_aggr_kernel():
    mesh = plsc.VectorSubcoreMesh(core_axis_name="c", subcore_axis_name="s")

    @functools.partial(
        pl.kernel,
        out_type=jax.ShapeDtypeStruct((NW, NPAD), jnp.float32),
        mesh=mesh,
        scratch_types=[
            pltpu.VMEM((FPW * NPAD,), jnp.float32),
            pltpu.VMEM((FPW * NPAD,), jnp.float32),
            pltpu.VMEM((2, CHUNK), jnp.int32),
            pltpu.VMEM((2, CHUNK), jnp.int32),
            pltpu.SemaphoreType.DMA,
            pltpu.SemaphoreType.DMA,
        ],
        compiler_params=pltpu.CompilerParams(
            needs_layout_passes=False, use_tc_tiling_on_sc=False),
    )
    def _edge_aggr(ed_hbm, xT_hbm, out_hbm, xv, av, eb0, eb1, sem0, sem1):
        wid = _wid()
        for f in range(FPW):
            pltpu.sync_copy(xT_hbm.at[wid * FPW + f],
                            xv.at[pl.ds(f * NPAD, NPAD)])

        @plsc.parallel_loop(0, FPW * NPAD // 16, 1, unroll=8)
        def _zero(i):
            av[pl.ds(i * 16, 16)] = jnp.zeros((16,), jnp.float32)

        def process(eb):
            @plsc.parallel_loop(0, CHUNK // 16, 2, unroll=4)
            def _grp(g):
                for u in range(2):
                    base = (g + u) * 16
                    sd = eb[0, pl.ds(base, 16)]
                    sg = lax.shift_right_logical(sd, 14)
                    dg = jnp.bitwise_and(sd, 16383)
                    wg = plsc.bitcast(eb[1, pl.ds(base, 16)], jnp.float32)
                    for f in range(FPW):
                        v = plsc.load_gather(xv, [sg + f * NPAD])
                        plsc.addupdate_scatter(av, [dg + f * NPAD], v * wg)

        pltpu.make_async_copy(ed_hbm.at[0], eb0, sem0).start()

        def pair(p, carry):
            ci = p * 2
            pltpu.make_async_copy(ed_hbm.at[ci], eb0, sem0).wait()
            pltpu.make_async_copy(ed_hbm.at[ci + 1], eb1, sem1).start()
            process(eb0)
            pltpu.make_async_copy(ed_hbm.at[ci + 1], eb1, sem1).wait()

            @pl.when(ci + 2 < NCH)
            def _():
                pltpu.make_async_copy(ed_hbm.at[ci + 2], eb0, sem0).start()

            process(eb1)
            return carry

        lax.fori_loop(0, NCH // 2, pair, 0)
        for f in range(FPW):
            pltpu.sync_copy(av.at[pl.ds(f * NPAD, NPAD)],
                            out_hbm.at[wid * FPW + f])

    return _edge_aggr


def _mask_vec():
    return (lax.broadcasted_iota(jnp.int32, (NPAD, 1), 0)
            < N).astype(jnp.float32)


def _d1_body(xT_ref, aggrT_ref, idxw_ref, Wrel_ref, Wroot_ref, vec_ref,
             out_ref):
    brel = vec_ref[0:NW, :]
    g = vec_ref[NW:2 * NW, :]
    b = vec_ref[2 * NW:3 * NW, :]
    xT = xT_ref[...]
    aggT = aggrT_ref[...] + xT * idxw_ref[...]
    hT = (jnp.dot(Wrel_ref[...], aggT, preferred_element_type=jnp.float32)
          + brel
          + jnp.dot(Wroot_ref[...], xT, preferred_element_type=jnp.float32))
    mvec = _mask_vec()
    mu = jnp.dot(hT, mvec, preferred_element_type=jnp.float32) / N
    dev = hT - mu
    var = jnp.dot(dev * dev, mvec, preferred_element_type=jnp.float32) / N
    hn = dev * lax.rsqrt(var + 1e-5) * g + b
    out_ref[0:NW, :] = jnp.tanh(hn)
    out_ref[NW:2 * NW, :] = xT


_CB = 2048


def _d23_body(e1_ref, W1_ref, W2_ref, vec_ref, xb_ref, out_ref):
    k = pl.program_id(0)
    b1 = vec_ref[0:NH, :]
    b2 = vec_ref[NH:2 * NH, :]
    e1 = e1_ref[...]
    a = jnp.dot(W1_ref[...], e1, preferred_element_type=jnp.float32) + b1
    c = jnp.dot(W2_ref[...], e1, preferred_element_type=jnp.float32) + b2
    e2 = jax.nn.sigmoid(a) * jnp.tanh(c)
    sel = (xb_ref[...].reshape(_CB, 1)
           == lax.broadcasted_iota(jnp.int32, (_CB, B), 1)).astype(jnp.float32)
    p = jnp.dot(e2, sel, preferred_element_type=jnp.float32)

    @pl.when(k == 0)
    def _():
        out_ref[...] = p

    @pl.when(k > 0)
    def _():
        out_ref[...] += p


def _d4_body(e1_ref, xb_ref, e2s_ref, Wm_ref, Wl_ref,
             Wpa_ref, Wpb_ref, vec_ref,
             meanT_ref, logvarT_ref, phiT_ref):
    bm = vec_ref[0:NT, :]
    bl = vec_ref[NT:2 * NT, :]
    bp = vec_ref[2 * NT:3 * NT, :]
    bg = vec_ref[3 * NT:4 * NT, :]
    bb = vec_ref[4 * NT:5 * NT, :]
    e2s = e2s_ref[...]
    meanT_ref[...] = (jnp.dot(Wm_ref[...], e2s, preferred_element_type=jnp.float32)
                      + bm)
    logvarT_ref[...] = (jnp.dot(Wl_ref[...], e2s, preferred_element_type=jnp.float32)
                        + bl)
    q = jnp.dot(Wpb_ref[...], e2s, preferred_element_type=jnp.float32)
    sel = (xb_ref[...].reshape(NPAD, 1)
           == lax.broadcasted_iota(jnp.int32, (NPAD, B), 1)).astype(jnp.float32)
    plT = (jnp.dot(Wpa_ref[...], e1_ref[...], preferred_element_type=jnp.float32)
           + lax.dot_general(q, sel, (((1,), (1,)), ((), ())),
                             preferred_element_type=jnp.float32)
           + bp)
    mvec = _mask_vec()
    mup = jnp.dot(plT, mvec, preferred_element_type=jnp.float32) / N
    dev = plT - mup
    varp = jnp.dot(dev * dev, mvec, preferred_element_type=jnp.float32) / N
    pln = dev * lax.rsqrt(varp + 1e-5) * bg + bb
    m = jnp.max(pln, axis=0, keepdims=True)
    ex = jnp.exp(pln - m)
    phiT_ref[...] = ex / jnp.sum(ex, axis=0, keepdims=True)


def kernel(idx_x, idx_w, x_batch, edge_index, edge_weight, word_vec, W_rel,
           b_rel, W_root, bn1_g, bn1_b, W_fc1, b_fc1, W_fc2, b_fc2, W_mean,
           b_mean, W_logvar, b_logvar, W_phi, b_phi, bnp_g, bnp_b):
    f32 = jnp.float32
    idx_pad = jnp.concatenate(
        [idx_x.astype(jnp.int32), jnp.zeros((NPAD - N,), jnp.int32)])
    x_pad = _vocab_gather_kernel()(idx_pad, word_vec)
    xT_pad = x_pad.T

    src = edge_index[0].astype(jnp.int32)
    dst = edge_index[1].astype(jnp.int32)
    ew_bits = lax.bitcast_convert_type(edge_weight, jnp.int32)
    ed = (jnp.stack([src * 16384 + dst, ew_bits], axis=0)
          .reshape(2, NCH, CHUNK).transpose(1, 0, 2))
    aggrT_pad = _edge_aggr_kernel()(ed, xT_pad)

    idxw_pad = jnp.pad(idx_w, (0, NPAD - N)).reshape(1, NPAD)
    xb_pad = jnp.pad(x_batch.astype(jnp.int32), (0, NPAD - N),
                     constant_values=B).reshape(1, NPAD)

    vec1 = jnp.concatenate([b_rel, bn1_g, bn1_b]).reshape(3 * NW, 1)
    enc1T = pl.pallas_call(
        _d1_body,
        out_shape=jax.ShapeDtypeStruct((2 * NW, NPAD), f32),
    )(xT_pad, aggrT_pad, idxw_pad, W_rel, W_root, vec1)

    ncb = NPAD // _CB
    vec2 = jnp.concatenate([b_fc1, b_fc2]).reshape(2 * NH, 1)
    enc2sT = pl.pallas_call(
        _d23_body,
        grid=(ncb,),
        in_specs=[
            pl.BlockSpec((2 * NW, _CB), lambda k: (0, k)),
            pl.BlockSpec((NH, 2 * NW), lambda k: (0, 0)),
            pl.BlockSpec((NH, 2 * NW), lambda k: (0, 0)),
            pl.BlockSpec((2 * NH, 1), lambda k: (0, 0)),
            pl.BlockSpec((1, _CB), lambda k: (0, k)),
        ],
        out_specs=pl.BlockSpec((NH, B), lambda k: (0, 0)),
        out_shape=jax.ShapeDtypeStruct((NH, B), f32),
    )(enc1T, W_fc1, W_fc2, vec2, xb_pad)

    vec4 = jnp.concatenate(
        [b_mean, b_logvar, b_phi, bnp_g, bnp_b]).reshape(5 * NT, 1)
    meanT, logvarT, phiT = pl.pallas_call(
        _d4_body,
        out_shape=(
            jax.ShapeDtypeStruct((NT, B), f32),
            jax.ShapeDtypeStruct((NT, B), f32),
            jax.ShapeDtypeStruct((NT, NPAD), f32),
        ),
    )(enc1T, xb_pad, enc2sT, W_mean, W_logvar,
      W_phi[:, :2 * NW], W_phi[:, 2 * NW:], vec4)

    return (meanT.T, logvarT.T, phiT[:, :N].T)


# speedup vs baseline: 1.0822x; 1.0009x over previous
import functools

import jax
import jax.numpy as jnp
from jax import lax
from jax.experimental import pallas as pl
from jax.experimental.pallas import tpu as pltpu
from jax.experimental.pallas import tpu_sc as plsc

N = 10000
E = 320000
VOCAB = 100000
NW = 128
NH = 512
NT = 128
B = 512

NWORK = 32
NPAD = 10240
RPW = NPAD // NWORK
FPW = NW // NWORK
CHUNK = 6400
NCH = E // CHUNK

def _wid():
    return lax.axis_index("s") * 2 + lax.axis_index("c")


@functools.cache
def _vocab_gather_kernel():
    mesh = plsc.VectorSubcoreMesh(core_axis_name="c", subcore_axis_name="s")

    @functools.partial(
        pl.kernel,
        out_type=jax.ShapeDtypeStruct((NPAD, NW), jnp.float32),
        mesh=mesh,
        scratch_types=[
            pltpu.VMEM((RPW,), jnp.int32),
            pltpu.VMEM((RPW, NW), jnp.float32),
            pltpu.SemaphoreType.DMA,
        ],
    )
    def _vocab_gather(idx_hbm, table_hbm, out_hbm, idx_v, rows_v, sem):
        base = _wid() * RPW
        pltpu.sync_copy(idx_hbm.at[pl.ds(base, RPW)], idx_v)
        pltpu.async_copy(table_hbm.at[idx_v], rows_v, sem).wait()
        pltpu.sync_copy(rows_v, out_hbm.at[pl.ds(base, RPW)])

    return _vocab_gather


@functools.cache
def _edge_aggr_kernel():
    mesh = plsc.VectorSubcoreMesh(core_axis_name="c", subcore_axis_name="s")

    @functools.partial(
        pl.kernel,
        out_type=jax.ShapeDtypeStruct((NW, NPAD), jnp.float32),
        mesh=mesh,
        scratch_types=[
            pltpu.VMEM((FPW * NPAD,), jnp.float32),
            pltpu.VMEM((FPW * NPAD,), jnp.float32),
            pltpu.VMEM((2, CHUNK), jnp.int32),
            pltpu.VMEM((2, CHUNK), jnp.int32),
            pltpu.SemaphoreType.DMA,
            pltpu.SemaphoreType.DMA,
        ],
        compiler_params=pltpu.CompilerParams(
            needs_layout_passes=False, use_tc_tiling_on_sc=False),
    )
    def _edge_aggr(ed_hbm, xT_hbm, out_hbm, xv, av, eb0, eb1, sem0, sem1):
        wid = _wid()
        for f in range(FPW):
            pltpu.sync_copy(xT_hbm.at[wid * FPW + f],
                            xv.at[pl.ds(f * NPAD, NPAD)])

        @plsc.parallel_loop(0, FPW * NPAD // 16, 1, unroll=8)
        def _zero(i):
            av[pl.ds(i * 16, 16)] = jnp.zeros((16,), jnp.float32)

        def process(eb):
            @plsc.parallel_loop(0, CHUNK // 16, 2, unroll=4)
            def _grp(g):
                for u in range(2):
                    base = (g + u) * 16
                    sd = eb[0, pl.ds(base, 16)]
                    sg = lax.shift_right_logical(sd, 14)
                    dg = jnp.bitwise_and(sd, 16383)
                    wg = plsc.bitcast(eb[1, pl.ds(base, 16)], jnp.float32)
                    for f in range(FPW):
                        v = plsc.load_gather(xv, [sg + f * NPAD])
                        plsc.addupdate_scatter(av, [dg + f * NPAD], v * wg)

        pltpu.make_async_copy(ed_hbm.at[0], eb0, sem0).start()

        def pair(p, carry):
            ci = p * 2
            pltpu.make_async_copy(ed_hbm.at[ci], eb0, sem0).wait()
            pltpu.make_async_copy(ed_hbm.at[ci + 1], eb1, sem1).start()
            process(eb0)
            pltpu.make_async_copy(ed_hbm.at[ci + 1], eb1, sem1).wait()

            @pl.when(ci + 2 < NCH)
            def _():
                pltpu.make_async_copy(ed_hbm.at[ci + 2], eb0, sem0).start()

            process(eb1)
            return carry

        lax.fori_loop(0, NCH // 2, pair, 0)
        for f in range(FPW):
            pltpu.sync_copy(av.at[pl.ds(f * NPAD, NPAD)],
                            out_hbm.at[wid * FPW + f])

    return _edge_aggr


def _mask_vec():
    return (lax.broadcasted_iota(jnp.int32, (NPAD, 1), 0)
            < N).astype(jnp.float32)


def _d1_body(xT_ref, aggrT_ref, idxw_ref, Wrel_ref, Wroot_ref, vec_ref,
             out_ref):
    brel = vec_ref[0:NW, :]
    g = vec_ref[NW:2 * NW, :]
    b = vec_ref[2 * NW:3 * NW, :]
    xT = xT_ref[...]
    aggT = aggrT_ref[...] + xT * idxw_ref[...]
    hT = (jnp.dot(Wrel_ref[...], aggT, preferred_element_type=jnp.float32)
          + brel
          + jnp.dot(Wroot_ref[...], xT, preferred_element_type=jnp.float32))
    mvec = _mask_vec()
    mu = jnp.dot(hT, mvec, preferred_element_type=jnp.float32) / N
    dev = hT - mu
    var = jnp.dot(dev * dev, mvec, preferred_element_type=jnp.float32) / N
    hn = dev * lax.rsqrt(var + 1e-5) * g + b
    out_ref[0:NW, :] = jnp.tanh(hn)
    out_ref[NW:2 * NW, :] = xT


_CB = 2048


def _d23_body(e1_ref, W1_ref, W2_ref, vec_ref, xb_ref, out_ref):
    k = pl.program_id(0)
    b1 = vec_ref[0:NH, :]
    b2 = vec_ref[NH:2 * NH, :]
    e1 = e1_ref[...]
    a = jnp.dot(W1_ref[...], e1, preferred_element_type=jnp.float32) + b1
    c = jnp.dot(W2_ref[...], e1, preferred_element_type=jnp.float32) + b2
    e2 = jax.nn.sigmoid(a) * jnp.tanh(c)
    sel = (xb_ref[...].reshape(_CB, 1)
           == lax.broadcasted_iota(jnp.int32, (_CB, B), 1)).astype(jnp.float32)
    p = jnp.dot(e2, sel, preferred_element_type=jnp.float32)

    @pl.when(k == 0)
    def _():
        out_ref[...] = p

    @pl.when(k > 0)
    def _():
        out_ref[...] += p


def _d4_body(e1_ref, xb_ref, e2s_ref, Wm_ref, Wl_ref,
             Wpa_ref, Wpb_ref, vec_ref,
             meanT_ref, logvarT_ref, phiT_ref):
    bm = vec_ref[0:NT, :]
    bl = vec_ref[NT:2 * NT, :]
    bp = vec_ref[2 * NT:3 * NT, :]
    bg = vec_ref[3 * NT:4 * NT, :]
    bb = vec_ref[4 * NT:5 * NT, :]
    e2s = e2s_ref[...]
    meanT_ref[...] = (jnp.dot(Wm_ref[...], e2s, preferred_element_type=jnp.float32)
                      + bm)
    logvarT_ref[...] = (jnp.dot(Wl_ref[...], e2s, preferred_element_type=jnp.float32)
                        + bl)
    q = jnp.dot(Wpb_ref[...], e2s, preferred_element_type=jnp.float32)
    sel = (xb_ref[...].reshape(NPAD, 1)
           == lax.broadcasted_iota(jnp.int32, (NPAD, B), 1)).astype(jnp.float32)
    plT = (jnp.dot(Wpa_ref[...], e1_ref[...], preferred_element_type=jnp.float32)
           + lax.dot_general(q, sel, (((1,), (1,)), ((), ())),
                             preferred_element_type=jnp.float32)
           + bp)
    mvec = _mask_vec()
    mup = jnp.dot(plT, mvec, preferred_element_type=jnp.float32) / N
    dev = plT - mup
    varp = jnp.dot(dev * dev, mvec, preferred_element_type=jnp.float32) / N
    pln = dev * lax.rsqrt(varp + 1e-5) * bg + bb
    m = jnp.max(pln, axis=0, keepdims=True)
    ex = jnp.exp(pln - m)
    phiT_ref[...] = ex / jnp.sum(ex, axis=0, keepdims=True)


def kernel(idx_x, idx_w, x_batch, edge_index, edge_weight, word_vec, W_rel,
           b_rel, W_root, bn1_g, bn1_b, W_fc1, b_fc1, W_fc2, b_fc2, W_mean,
           b_mean, W_logvar, b_logvar, W_phi, b_phi, bnp_g, bnp_b):
    f32 = jnp.float32
    idx_pad = jnp.concatenate(
        [idx_x.astype(jnp.int32), jnp.zeros((NPAD - N,), jnp.int32)])
    x_pad = _vocab_gather_kernel()(idx_pad, word_vec)
    xT_pad = x_pad.T

    src = edge_index[0].astype(jnp.int32)
    dst = edge_index[1].astype(jnp.int32)
    ew_bits = lax.bitcast_convert_type(edge_weight, jnp.int32)
    ed = (jnp.stack([src * 16384 + dst, ew_bits], axis=0)
          .reshape(2, NCH, CHUNK).transpose(1, 0, 2))
    aggrT_pad = _edge_aggr_kernel()(ed, xT_pad)

    idxw_pad = jnp.pad(idx_w, (0, NPAD - N)).reshape(1, NPAD)
    xb_pad = jnp.pad(x_batch.astype(jnp.int32), (0, NPAD - N),
                     constant_values=B).reshape(1, NPAD)

    vec1 = jnp.concatenate([b_rel, bn1_g, bn1_b]).reshape(3 * NW, 1)
    enc1T = pl.pallas_call(
        _d1_body,
        out_shape=jax.ShapeDtypeStruct((2 * NW, NPAD), f32),
    )(xT_pad, aggrT_pad, idxw_pad, W_rel, W_root, vec1)

    ncb = NPAD // _CB
    vec2 = jnp.concatenate([b_fc1, b_fc2]).reshape(2 * NH, 1)
    enc2sT = pl.pallas_call(
        _d23_body,
        grid=(ncb,),
        in_specs=[
            pl.BlockSpec((2 * NW, _CB), lambda k: (0, k)),
            pl.BlockSpec((NH, 2 * NW), lambda k: (0, 0)),
            pl.BlockSpec((NH, 2 * NW), lambda k: (0, 0)),
            pl.BlockSpec((2 * NH, 1), lambda k: (0, 0)),
            pl.BlockSpec((1, _CB), lambda k: (0, k)),
        ],
        out_specs=pl.BlockSpec((NH, B), lambda k: (0, 0)),
        out_shape=jax.ShapeDtypeStruct((NH, B), f32),
    )(enc1T, W_fc1, W_fc2, vec2, xb_pad)

    vec4 = jnp.concatenate(
        [b_mean, b_logvar, b_phi, bnp_g, bnp_b]).reshape(5 * NT, 1)
    meanT, logvarT, phiT = pl.pallas_call(
        _d4_body,
        out_shape=(
            jax.ShapeDtypeStruct((NT, B), f32),
            jax.ShapeDtypeStruct((NT, B), f32),
            jax.ShapeDtypeStruct((NT, NPAD), f32),
        ),
    )(enc1T, xb_pad, enc2sT, W_mean, W_logvar,
      W_phi[:, :2 * NW], W_phi[:, 2 * NW:], vec4)

    return (meanT.T, logvarT.T, phiT[:, :N].T)
